# baseline (speedup 1.0000x reference)
import functools

import jax
import jax.numpy as jnp
from jax.experimental import pallas as pl

NUM_EXPERTS = 64
NUM_HASH = 4
HIDDEN = 768
HIDDEN_Q = HIDDEN // 4


def _routing_kernel(x_ref, w1_ref, b1_ref, w2bd_ref, b2_ref,
                    rw_ref, ea_ref, hash_ref, lb_ref):
    x = x_ref[...].astype(jnp.bfloat16)
    h = jnp.dot(x, w1_ref[...].astype(jnp.bfloat16),
                preferred_element_type=jnp.float32)
    h = jnp.maximum(h + b1_ref[...], 0.0)
    hv = jnp.dot(h.astype(jnp.bfloat16), w2bd_ref[...].astype(jnp.bfloat16),
                 preferred_element_type=jnp.float32)
    hv = hv + b2_ref[...]
    hash_ref[...] = hv
    summed = jnp.sum(hv, axis=1)
    e = summed.astype(jnp.int32)
    r = jnp.bitwise_and(e, NUM_EXPERTS - 1)
    ea_ref[...] = r
    lanes = jax.lax.broadcasted_iota(jnp.int32, (x.shape[0], NUM_EXPERTS), 1)
    rw = (lanes == r[:, None]).astype(jnp.float32)
    rw_ref[...] = rw

    @pl.when(pl.program_id(0) == 0)
    def _init():
        lb_ref[...] = jnp.zeros_like(lb_ref)

    lb_ref[...] += jnp.sum(rw, axis=0)


@functools.partial(jax.jit, static_argnames=("block_t",))
def _run(hidden_flat, W1cat, b1cat, W2bd, b2row, block_t):
    T = hidden_flat.shape[0]
    grid = (T // block_t,)
    rw, ea, hashes, lb = pl.pallas_call(
        _routing_kernel,
        grid=grid,
        in_specs=[
            pl.BlockSpec((block_t, HIDDEN), lambda i: (i, 0)),
            pl.BlockSpec((HIDDEN, HIDDEN), lambda i: (0, 0)),
            pl.BlockSpec((1, HIDDEN), lambda i: (0, 0)),
            pl.BlockSpec((HIDDEN, NUM_HASH), lambda i: (0, 0)),
            pl.BlockSpec((1, NUM_HASH), lambda i: (0, 0)),
        ],
        out_specs=[
            pl.BlockSpec((block_t, NUM_EXPERTS), lambda i: (i, 0)),
            pl.BlockSpec((block_t,), lambda i: (i,)),
            pl.BlockSpec((block_t, NUM_HASH), lambda i: (i, 0)),
            pl.BlockSpec((NUM_EXPERTS,), lambda i: (0,)),
        ],
        out_shape=[
            jax.ShapeDtypeStruct((T, NUM_EXPERTS), jnp.float32),
            jax.ShapeDtypeStruct((T,), jnp.int32),
            jax.ShapeDtypeStruct((T, NUM_HASH), jnp.float32),
            jax.ShapeDtypeStruct((NUM_EXPERTS,), jnp.float32),
        ],
    )(hidden_flat, W1cat, b1cat, W2bd, b2row)
    return rw, ea, hashes, lb


def kernel(hidden_states, W1, b1, W2, b2):
    B, S, H = hidden_states.shape
    hidden_flat = hidden_states.reshape(-1, H)
    W1cat = jnp.transpose(W1, (1, 0, 2)).reshape(H, NUM_HASH * HIDDEN_Q)
    b1cat = b1.reshape(1, NUM_HASH * HIDDEN_Q)
    eye = jnp.eye(NUM_HASH, dtype=jnp.float32)
    W2bd = (W2[:, :, 0][:, :, None] * eye[:, None, :]).reshape(
        NUM_HASH * HIDDEN_Q, NUM_HASH)
    b2row = b2[:, 0].reshape(1, NUM_HASH)
    rw, ea, hashes, lb = _run(hidden_flat, W1cat, b1cat, W2bd, b2row,
                              block_t=1024)
    return rw, ea[:, None], hashes, lb


# speedup vs baseline: 1.1693x; 1.1693x over previous
import functools

import jax
import jax.numpy as jnp
from jax.experimental import pallas as pl

NUM_EXPERTS = 64
NUM_HASH = 4
HIDDEN = 768
HIDDEN_Q = HIDDEN // 4


def _routing_kernel(x_ref, w1_ref, b1_ref, w2bd_ref, b2_ref,
                    rw_ref, ea_ref, hash_ref, lb_ref):
    x = x_ref[...].astype(jnp.bfloat16)
    h = jnp.dot(x, w1_ref[...], preferred_element_type=jnp.float32)
    h = jnp.maximum(h + b1_ref[...], 0.0)
    hv = jnp.dot(h.astype(jnp.bfloat16), w2bd_ref[...],
                 preferred_element_type=jnp.float32)
    hv = hv + b2_ref[...]
    hash_ref[...] = hv
    summed = jnp.sum(hv, axis=1, keepdims=True)
    e = summed.astype(jnp.int32)
    r = jnp.bitwise_and(e, NUM_EXPERTS - 1)
    ea_ref[...] = r
    lanes = jax.lax.broadcasted_iota(jnp.int32, (x.shape[0], NUM_EXPERTS), 1)
    rw = (lanes == r).astype(jnp.float32)
    rw_ref[...] = rw

    @pl.when(pl.program_id(0) == 0)
    def _init():
        lb_ref[...] = jnp.zeros_like(lb_ref)

    lb_ref[...] += jnp.sum(rw, axis=0)


@functools.partial(jax.jit, static_argnames=("block_t",))
def _run(hidden_flat, W1cat, b1cat, W2bd, b2row, block_t):
    T = hidden_flat.shape[0]
    grid = (T // block_t,)
    rw, ea, hashes, lb = pl.pallas_call(
        _routing_kernel,
        grid=grid,
        in_specs=[
            pl.BlockSpec((block_t, HIDDEN), lambda i: (i, 0)),
            pl.BlockSpec((HIDDEN, HIDDEN), lambda i: (0, 0)),
            pl.BlockSpec((1, HIDDEN), lambda i: (0, 0)),
            pl.BlockSpec((HIDDEN, NUM_HASH), lambda i: (0, 0)),
            pl.BlockSpec((1, NUM_HASH), lambda i: (0, 0)),
        ],
        out_specs=[
            pl.BlockSpec((block_t, NUM_EXPERTS), lambda i: (i, 0)),
            pl.BlockSpec((block_t, 1), lambda i: (i, 0)),
            pl.BlockSpec((block_t, NUM_HASH), lambda i: (i, 0)),
            pl.BlockSpec((NUM_EXPERTS,), lambda i: (0,)),
        ],
        out_shape=[
            jax.ShapeDtypeStruct((T, NUM_EXPERTS), jnp.float32),
            jax.ShapeDtypeStruct((T, 1), jnp.int32),
            jax.ShapeDtypeStruct((T, NUM_HASH), jnp.float32),
            jax.ShapeDtypeStruct((NUM_EXPERTS,), jnp.float32),
        ],
    )(hidden_flat, W1cat, b1cat, W2bd, b2row)
    return rw, ea, hashes, lb


def kernel(hidden_states, W1, b1, W2, b2):
    B, S, H = hidden_states.shape
    hidden_flat = hidden_states.reshape(-1, H)
    W1cat = jnp.transpose(W1, (1, 0, 2)).reshape(
        H, NUM_HASH * HIDDEN_Q).astype(jnp.bfloat16)
    b1cat = b1.reshape(1, NUM_HASH * HIDDEN_Q)
    eye = jnp.eye(NUM_HASH, dtype=jnp.float32)
    W2bd = (W2[:, :, 0][:, :, None] * eye[:, None, :]).reshape(
        NUM_HASH * HIDDEN_Q, NUM_HASH).astype(jnp.bfloat16)
    b2row = b2[:, 0].reshape(1, NUM_HASH)
    rw, ea, hashes, lb = _run(hidden_flat, W1cat, b1cat, W2bd, b2row,
                              block_t=1024)
    return rw, ea, hashes, lb


# speedup vs baseline: 1.1753x; 1.0052x over previous
import functools

import jax
import jax.numpy as jnp
from jax.experimental import pallas as pl
from jax.experimental.pallas import tpu as pltpu

NUM_EXPERTS = 64
NUM_HASH = 4
HIDDEN = 768
HIDDEN_Q = HIDDEN // 4


def _routing_kernel(x_ref, w1_ref, b1_ref, w2_ref, b2_ref,
                    rw_ref, ea_ref, hash_ref, lb_ref,
                    w1s_ref, w2s_ref):
    @pl.when(pl.program_id(0) == 0)
    def _prep():
        w2s_ref[...] = jnp.zeros_like(w2s_ref)
        for k in range(NUM_HASH):
            w1s_ref[:, k * HIDDEN_Q:(k + 1) * HIDDEN_Q] = (
                w1_ref[k].astype(jnp.bfloat16))
            w2s_ref[k * HIDDEN_Q:(k + 1) * HIDDEN_Q, k:k + 1] = (
                w2_ref[k].astype(jnp.bfloat16))
        lb_ref[...] = jnp.zeros_like(lb_ref)

    x = x_ref[...].astype(jnp.bfloat16)
    h = jnp.dot(x, w1s_ref[...], preferred_element_type=jnp.float32)
    h = jnp.maximum(h + b1_ref[...], 0.0)
    hv = jnp.dot(h.astype(jnp.bfloat16), w2s_ref[...],
                 preferred_element_type=jnp.float32)
    hv = hv + b2_ref[...]
    hash_ref[...] = hv
    summed = jnp.sum(hv, axis=1, keepdims=True)
    e = summed.astype(jnp.int32)
    r = jnp.bitwise_and(e, NUM_EXPERTS - 1)
    ea_ref[...] = r
    lanes = jax.lax.broadcasted_iota(jnp.int32, (x.shape[0], NUM_EXPERTS), 1)
    rw = (lanes == r).astype(jnp.float32)
    rw_ref[...] = rw
    lb_ref[...] += jnp.sum(rw, axis=0)


@functools.partial(jax.jit, static_argnames=("block_t",))
def _run(hidden_flat, W1, b1cat, W2, b2row, block_t):
    T = hidden_flat.shape[0]
    grid = (T // block_t,)
    rw, ea, hashes, lb = pl.pallas_call(
        _routing_kernel,
        grid=grid,
        in_specs=[
            pl.BlockSpec((block_t, HIDDEN), lambda i: (i, 0)),
            pl.BlockSpec((NUM_HASH, HIDDEN, HIDDEN_Q), lambda i: (0, 0, 0)),
            pl.BlockSpec((1, HIDDEN), lambda i: (0, 0)),
            pl.BlockSpec((NUM_HASH, HIDDEN_Q, 1), lambda i: (0, 0, 0)),
            pl.BlockSpec((1, NUM_HASH), lambda i: (0, 0)),
        ],
        out_specs=[
            pl.BlockSpec((block_t, NUM_EXPERTS), lambda i: (i, 0)),
            pl.BlockSpec((block_t, 1), lambda i: (i, 0)),
            pl.BlockSpec((block_t, NUM_HASH), lambda i: (i, 0)),
            pl.BlockSpec((NUM_EXPERTS,), lambda i: (0,)),
        ],
        out_shape=[
            jax.ShapeDtypeStruct((T, NUM_EXPERTS), jnp.float32),
            jax.ShapeDtypeStruct((T, 1), jnp.int32),
            jax.ShapeDtypeStruct((T, NUM_HASH), jnp.float32),
            jax.ShapeDtypeStruct((NUM_EXPERTS,), jnp.float32),
        ],
        scratch_shapes=[
            pltpu.VMEM((HIDDEN, NUM_HASH * HIDDEN_Q), jnp.bfloat16),
            pltpu.VMEM((NUM_HASH * HIDDEN_Q, NUM_HASH), jnp.bfloat16),
        ],
    )(hidden_flat, W1, b1cat, W2, b2row)
    return rw, ea, hashes, lb


def kernel(hidden_states, W1, b1, W2, b2):
    B, S, H = hidden_states.shape
    hidden_flat = hidden_states.reshape(-1, H)
    b1cat = b1.reshape(1, NUM_HASH * HIDDEN_Q)
    b2row = b2.reshape(1, NUM_HASH)
    rw, ea, hashes, lb = _run(hidden_flat, W1, b1cat, W2, b2row,
                              block_t=1024)
    return rw, ea, hashes, lb


# speedup vs baseline: 1.6994x; 1.4459x over previous
import functools

import jax
import jax.numpy as jnp
from jax.experimental import pallas as pl
from jax.experimental.pallas import tpu as pltpu

NUM_EXPERTS = 64
NUM_HASH = 4
HIDDEN = 768
HIDDEN_Q = HIDDEN // 4


def _routing_kernel(x_ref, w1_ref, b1_ref, w2_ref, b2_ref,
                    rw_ref, ea_ref, hash_ref, lb_ref,
                    w1s_ref, w2s_ref):
    @pl.when(pl.program_id(0) == 0)
    def _prep():
        w2s_ref[...] = jnp.zeros_like(w2s_ref)
        for k in range(NUM_HASH):
            w1s_ref[:, k * HIDDEN_Q:(k + 1) * HIDDEN_Q] = (
                w1_ref[k].astype(jnp.bfloat16))
            w2s_ref[k * HIDDEN_Q:(k + 1) * HIDDEN_Q, k:k + 1] = (
                w2_ref[k].astype(jnp.bfloat16))
        lb_ref[...] = jnp.zeros_like(lb_ref)

    x = x_ref[...].astype(jnp.bfloat16)
    h = jnp.dot(x, w1s_ref[...], preferred_element_type=jnp.float32)
    h = jnp.maximum(h + b1_ref[...], 0.0)
    hv = jnp.dot(h.astype(jnp.bfloat16), w2s_ref[...],
                 preferred_element_type=jnp.float32)
    hv = hv + b2_ref[...]
    hvT = jnp.transpose(hv)
    hash_ref[...] = hvT
    summed = jnp.sum(hvT, axis=0, keepdims=True)
    e = summed.astype(jnp.int32)
    r = jnp.bitwise_and(e, NUM_EXPERTS - 1)
    ea_ref[...] = r.reshape(r.shape[1])
    subl = jax.lax.broadcasted_iota(
        jnp.int32, (NUM_EXPERTS, r.shape[1]), 0)
    rwT = (subl == r).astype(jnp.float32)
    rw_ref[...] = rwT
    lb_ref[...] += jnp.sum(rwT, axis=1, keepdims=True)


@functools.partial(jax.jit, static_argnames=("block_t",))
def _run(hidden_flat, W1, b1cat, W2, b2row, block_t):
    T = hidden_flat.shape[0]
    grid = (T // block_t,)
    rwT, ea, hashesT, lb = pl.pallas_call(
        _routing_kernel,
        grid=grid,
        in_specs=[
            pl.BlockSpec((block_t, HIDDEN), lambda i: (i, 0)),
            pl.BlockSpec((NUM_HASH, HIDDEN, HIDDEN_Q), lambda i: (0, 0, 0)),
            pl.BlockSpec((1, HIDDEN), lambda i: (0, 0)),
            pl.BlockSpec((NUM_HASH, HIDDEN_Q, 1), lambda i: (0, 0, 0)),
            pl.BlockSpec((1, NUM_HASH), lambda i: (0, 0)),
        ],
        out_specs=[
            pl.BlockSpec((NUM_EXPERTS, block_t), lambda i: (0, i)),
            pl.BlockSpec((block_t,), lambda i: (i,)),
            pl.BlockSpec((NUM_HASH, block_t), lambda i: (0, i)),
            pl.BlockSpec((NUM_EXPERTS, 1), lambda i: (0, 0)),
        ],
        out_shape=[
            jax.ShapeDtypeStruct((NUM_EXPERTS, T), jnp.float32),
            jax.ShapeDtypeStruct((T,), jnp.int32),
            jax.ShapeDtypeStruct((NUM_HASH, T), jnp.float32),
            jax.ShapeDtypeStruct((NUM_EXPERTS, 1), jnp.float32),
        ],
        scratch_shapes=[
            pltpu.VMEM((HIDDEN, NUM_HASH * HIDDEN_Q), jnp.bfloat16),
            pltpu.VMEM((NUM_HASH * HIDDEN_Q, NUM_HASH), jnp.bfloat16),
        ],
    )(hidden_flat, W1, b1cat, W2, b2row)
    return rwT, ea, hashesT, lb


def kernel(hidden_states, W1, b1, W2, b2):
    B, S, H = hidden_states.shape
    hidden_flat = hidden_states.reshape(-1, H)
    b1cat = b1.reshape(1, NUM_HASH * HIDDEN_Q)
    b2row = b2.reshape(1, NUM_HASH)
    rwT, ea, hashesT, lb = _run(hidden_flat, W1, b1cat, W2, b2row,
                                block_t=1024)
    return rwT.T, ea[:, None], hashesT.T, lb.reshape(NUM_EXPERTS)


# speedup vs baseline: 1.7574x; 1.0341x over previous
import functools

import jax
import jax.numpy as jnp
from jax.experimental import pallas as pl
from jax.experimental.pallas import tpu as pltpu

NUM_EXPERTS = 64
NUM_HASH = 4
HIDDEN = 768
HIDDEN_Q = HIDDEN // 4


def _routing_kernel(x_ref, w1t_ref, pack_ref,
                    rw_ref, ea_ref, hash_ref, lb_ref,
                    w1s_ref, w2s_ref, b2s_ref):
    @pl.when(pl.program_id(0) == 0)
    def _prep():
        for k in range(NUM_HASH):
            w1s_ref[:, k * HIDDEN_Q:(k + 1) * HIDDEN_Q] = (
                w1t_ref[k].T.astype(jnp.bfloat16))
        w2col = jnp.transpose(pack_ref[0:1, :])
        rows = jax.lax.broadcasted_iota(jnp.int32, (HIDDEN, NUM_HASH), 0)
        cols = jax.lax.broadcasted_iota(jnp.int32, (HIDDEN, NUM_HASH), 1)
        w2s_ref[...] = jnp.where(rows // HIDDEN_Q == cols, w2col,
                                 0.0).astype(jnp.bfloat16)
        b2s_ref[...] = jnp.transpose(pack_ref[2:3, :NUM_HASH])
        lb_ref[...] = jnp.zeros_like(lb_ref)

    x = x_ref[...].astype(jnp.bfloat16)
    h = jnp.dot(x, w1s_ref[...], preferred_element_type=jnp.float32)
    h = jnp.maximum(h + pack_ref[1:2, :], 0.0)
    hv = jnp.dot(h.astype(jnp.bfloat16), w2s_ref[...],
                 preferred_element_type=jnp.float32)
    hvT = jnp.transpose(hv) + b2s_ref[...]
    hash_ref[...] = hvT
    summed = jnp.sum(hvT, axis=0, keepdims=True)
    e = summed.astype(jnp.int32)
    r = jnp.bitwise_and(e, NUM_EXPERTS - 1)
    ea_ref[...] = r.reshape(r.shape[1])
    subl = jax.lax.broadcasted_iota(
        jnp.int32, (NUM_EXPERTS, r.shape[1]), 0)
    rwT = (subl == r).astype(jnp.float32)
    rw_ref[...] = rwT
    lb_ref[...] += jnp.sum(rwT, axis=1)


@functools.partial(jax.jit, static_argnames=("block_t",))
def _run(hidden_flat, W1t, pack, block_t):
    T = hidden_flat.shape[0]
    grid = (T // block_t,)
    rwT, ea, hashesT, lb = pl.pallas_call(
        _routing_kernel,
        grid=grid,
        in_specs=[
            pl.BlockSpec((block_t, HIDDEN), lambda i: (i, 0)),
            pl.BlockSpec((NUM_HASH, HIDDEN_Q, HIDDEN), lambda i: (0, 0, 0)),
            pl.BlockSpec((3, HIDDEN), lambda i: (0, 0)),
        ],
        out_specs=[
            pl.BlockSpec((NUM_EXPERTS, block_t), lambda i: (0, i)),
            pl.BlockSpec((block_t,), lambda i: (i,)),
            pl.BlockSpec((NUM_HASH, block_t), lambda i: (0, i)),
            pl.BlockSpec((NUM_EXPERTS,), lambda i: (0,)),
        ],
        out_shape=[
            jax.ShapeDtypeStruct((NUM_EXPERTS, T), jnp.float32),
            jax.ShapeDtypeStruct((T,), jnp.int32),
            jax.ShapeDtypeStruct((NUM_HASH, T), jnp.float32),
            jax.ShapeDtypeStruct((NUM_EXPERTS,), jnp.float32),
        ],
        scratch_shapes=[
            pltpu.VMEM((HIDDEN, NUM_HASH * HIDDEN_Q), jnp.bfloat16),
            pltpu.VMEM((NUM_HASH * HIDDEN_Q, NUM_HASH), jnp.bfloat16),
            pltpu.VMEM((NUM_HASH, 1), jnp.float32),
        ],
    )(hidden_flat, W1t, pack)
    return rwT, ea, hashesT, lb


def kernel(hidden_states, W1, b1, W2, b2):
    B, S, H = hidden_states.shape
    hidden_flat = hidden_states.reshape(-1, H)
    W1t = jnp.transpose(W1, (0, 2, 1))
    pack = jnp.concatenate([
        W2.reshape(1, H),
        b1.reshape(1, H),
        jnp.pad(b2.reshape(1, NUM_HASH), ((0, 0), (0, H - NUM_HASH))),
    ], axis=0)
    rwT, ea, hashesT, lb = _run(hidden_flat, W1t, pack, block_t=1024)
    return rwT.T, ea[:, None], hashesT.T, lb


# speedup vs baseline: 1.9510x; 1.1102x over previous
import functools

import jax
import jax.numpy as jnp
from jax.experimental import pallas as pl
from jax.experimental.pallas import tpu as pltpu

NUM_EXPERTS = 64
NUM_HASH = 4
HIDDEN = 768
HIDDEN_Q = HIDDEN // 4


def _routing_kernel(x_ref, w1t_ref, pack_ref,
                    rw_ref, ea_ref, hash_ref, lb_ref,
                    w1s_ref, w2s_ref, b2s_ref):
    @pl.when(pl.program_id(0) == 0)
    def _prep():
        for k in range(NUM_HASH):
            w1s_ref[:, k * HIDDEN_Q:(k + 1) * HIDDEN_Q] = (
                w1t_ref[k].T.astype(jnp.bfloat16))
        w2col = jnp.transpose(pack_ref[0:1, :])
        rows = jax.lax.broadcasted_iota(jnp.int32, (HIDDEN, NUM_HASH), 0)
        cols = jax.lax.broadcasted_iota(jnp.int32, (HIDDEN, NUM_HASH), 1)
        w2s_ref[...] = jnp.where(rows // HIDDEN_Q == cols, w2col,
                                 0.0).astype(jnp.bfloat16)
        b2s_ref[...] = jnp.transpose(pack_ref[2:3, :NUM_HASH])
        lb_ref[...] = jnp.zeros_like(lb_ref)

    x = x_ref[...].astype(jnp.bfloat16)
    h = jnp.dot(x, w1s_ref[...], preferred_element_type=jnp.float32)
    h = jnp.maximum(h + pack_ref[1:2, :], 0.0)
    hv = jnp.dot(h.astype(jnp.bfloat16), w2s_ref[...],
                 preferred_element_type=jnp.float32)
    hvT = jnp.transpose(hv) + b2s_ref[...]
    hash_ref[...] = hvT
    summed = jnp.sum(hvT, axis=0, keepdims=True)
    e = summed.astype(jnp.int32)
    r = jnp.bitwise_and(e, NUM_EXPERTS - 1)
    ea_ref[...] = r.reshape(r.shape[1])
    subl = jax.lax.broadcasted_iota(
        jnp.int32, (NUM_EXPERTS, r.shape[1]), 0)
    rwT = (subl == r).astype(jnp.float32)
    rw_ref[...] = rwT
    lb_ref[...] += jnp.sum(rwT, axis=1)


@functools.partial(jax.jit, static_argnames=("block_t",))
def _run(hidden_flat, W1t, pack, block_t):
    T = hidden_flat.shape[0]
    grid = (T // block_t,)
    rwT, ea, hashesT, lb = pl.pallas_call(
        _routing_kernel,
        grid=grid,
        in_specs=[
            pl.BlockSpec((block_t, HIDDEN), lambda i: (i, 0)),
            pl.BlockSpec((NUM_HASH, HIDDEN_Q, HIDDEN), lambda i: (0, 0, 0)),
            pl.BlockSpec((3, HIDDEN), lambda i: (0, 0)),
        ],
        out_specs=[
            pl.BlockSpec((NUM_EXPERTS, block_t), lambda i: (0, i)),
            pl.BlockSpec((block_t,), lambda i: (i,)),
            pl.BlockSpec((NUM_HASH, block_t), lambda i: (0, i)),
            pl.BlockSpec((NUM_EXPERTS,), lambda i: (0,)),
        ],
        out_shape=[
            jax.ShapeDtypeStruct((NUM_EXPERTS, T), jnp.float32),
            jax.ShapeDtypeStruct((T,), jnp.int32),
            jax.ShapeDtypeStruct((NUM_HASH, T), jnp.float32),
            jax.ShapeDtypeStruct((NUM_EXPERTS,), jnp.float32),
        ],
        scratch_shapes=[
            pltpu.VMEM((HIDDEN, NUM_HASH * HIDDEN_Q), jnp.bfloat16),
            pltpu.VMEM((NUM_HASH * HIDDEN_Q, NUM_HASH), jnp.bfloat16),
            pltpu.VMEM((NUM_HASH, 1), jnp.float32),
        ],
    )(hidden_flat, W1t, pack)
    return rwT, ea, hashesT, lb


def kernel(hidden_states, W1, b1, W2, b2):
    B, S, H = hidden_states.shape
    hidden_flat = hidden_states.reshape(-1, H)
    W1t = jnp.transpose(W1, (0, 2, 1))
    pack = jnp.concatenate([
        W2.reshape(1, H),
        b1.reshape(1, H),
        jnp.pad(b2.reshape(1, NUM_HASH), ((0, 0), (0, H - NUM_HASH))),
    ], axis=0)
    rwT, ea, hashesT, lb = _run(hidden_flat, W1t, pack, block_t=2048)
    return rwT.T, ea[:, None], hashesT.T, lb


# speedup vs baseline: 2.0236x; 1.0372x over previous
import functools

import jax
import jax.numpy as jnp
from jax.experimental import pallas as pl
from jax.experimental.pallas import tpu as pltpu

NUM_EXPERTS = 64
NUM_HASH = 4
HIDDEN = 768
HIDDEN_Q = HIDDEN // 4


def _routing_kernel(x_ref, w1t_ref, pack_ref,
                    rw_ref, ea_ref, hash_ref, lb_ref,
                    w1s_ref, w2s_ref, b2s_ref):
    @pl.when(pl.program_id(0) == 0)
    def _prep():
        for k in range(NUM_HASH):
            w1s_ref[:, k * HIDDEN_Q:(k + 1) * HIDDEN_Q] = (
                w1t_ref[k].T.astype(jnp.bfloat16))
        w2col = jnp.transpose(pack_ref[0:1, :])
        rows = jax.lax.broadcasted_iota(jnp.int32, (HIDDEN, NUM_HASH), 0)
        cols = jax.lax.broadcasted_iota(jnp.int32, (HIDDEN, NUM_HASH), 1)
        w2s_ref[...] = jnp.where(rows // HIDDEN_Q == cols, w2col,
                                 0.0).astype(jnp.bfloat16)
        b2s_ref[...] = jnp.transpose(pack_ref[2:3, :NUM_HASH])
        lb_ref[...] = jnp.zeros_like(lb_ref)

    x = x_ref[...].astype(jnp.bfloat16)
    h = jnp.dot(x, w1s_ref[...], preferred_element_type=jnp.float32)
    h = jnp.maximum(h + pack_ref[1:2, :], 0.0)
    hv = jnp.dot(h.astype(jnp.bfloat16), w2s_ref[...],
                 preferred_element_type=jnp.float32)
    hvT = jnp.transpose(hv) + b2s_ref[...]
    hash_ref[...] = hvT
    summed = jnp.sum(hvT, axis=0, keepdims=True)
    e = summed.astype(jnp.int32)
    r = jnp.bitwise_and(e, NUM_EXPERTS - 1)
    ea_ref[...] = r.reshape(r.shape[1])
    subl = jax.lax.broadcasted_iota(
        jnp.int32, (NUM_EXPERTS, r.shape[1]), 0)
    rwT = (subl == r).astype(jnp.float32)
    rw_ref[...] = rwT
    lb_ref[...] += jnp.sum(rwT, axis=1)


@functools.partial(jax.jit, static_argnames=("block_t",))
def _run(hidden_flat, W1t, pack, block_t):
    T = hidden_flat.shape[0]
    grid = (T // block_t,)
    rwT, ea, hashesT, lb = pl.pallas_call(
        _routing_kernel,
        grid=grid,
        in_specs=[
            pl.BlockSpec((block_t, HIDDEN), lambda i: (i, 0)),
            pl.BlockSpec((NUM_HASH, HIDDEN_Q, HIDDEN), lambda i: (0, 0, 0)),
            pl.BlockSpec((3, HIDDEN), lambda i: (0, 0)),
        ],
        out_specs=[
            pl.BlockSpec((NUM_EXPERTS, block_t), lambda i: (0, i)),
            pl.BlockSpec((block_t,), lambda i: (i,)),
            pl.BlockSpec((NUM_HASH, block_t), lambda i: (0, i)),
            pl.BlockSpec((NUM_EXPERTS,), lambda i: (0,)),
        ],
        out_shape=[
            jax.ShapeDtypeStruct((NUM_EXPERTS, T), jnp.float32),
            jax.ShapeDtypeStruct((T,), jnp.int32),
            jax.ShapeDtypeStruct((NUM_HASH, T), jnp.float32),
            jax.ShapeDtypeStruct((NUM_EXPERTS,), jnp.float32),
        ],
        scratch_shapes=[
            pltpu.VMEM((HIDDEN, NUM_HASH * HIDDEN_Q), jnp.bfloat16),
            pltpu.VMEM((NUM_HASH * HIDDEN_Q, NUM_HASH), jnp.bfloat16),
            pltpu.VMEM((NUM_HASH, 1), jnp.float32),
        ],
    )(hidden_flat, W1t, pack)
    return rwT, ea, hashesT, lb


def kernel(hidden_states, W1, b1, W2, b2):
    B, S, H = hidden_states.shape
    hidden_flat = hidden_states.reshape(-1, H)
    W1t = jnp.transpose(W1, (0, 2, 1))
    pack = jnp.concatenate([
        W2.reshape(1, H),
        b1.reshape(1, H),
        jnp.pad(b2.reshape(1, NUM_HASH), ((0, 0), (0, H - NUM_HASH))),
    ], axis=0)
    rwT, ea, hashesT, lb = _run(hidden_flat, W1t, pack, block_t=4096)
    return rwT.T, ea[:, None], hashesT.T, lb


# speedup vs baseline: 2.0538x; 1.0149x over previous
import functools

import jax
import jax.numpy as jnp
from jax.experimental import pallas as pl
from jax.experimental.pallas import tpu as pltpu

NUM_EXPERTS = 64
NUM_HASH = 4
HIDDEN = 768
HIDDEN_Q = HIDDEN // 4
_CHUNK_M = 1024


def _routing_kernel(x_ref, w1t_ref, pack_ref,
                    rw_ref, ea_ref, hash_ref, lb_ref,
                    w1s_ref, w2s_ref, b2s_ref):
    @pl.when(pl.program_id(0) == 0)
    def _prep():
        for k in range(NUM_HASH):
            w1s_ref[:, k * HIDDEN_Q:(k + 1) * HIDDEN_Q] = (
                w1t_ref[k].T.astype(jnp.bfloat16))
        w2col = jnp.transpose(pack_ref[0:1, :])
        rows = jax.lax.broadcasted_iota(jnp.int32, (HIDDEN, NUM_HASH), 0)
        cols = jax.lax.broadcasted_iota(jnp.int32, (HIDDEN, NUM_HASH), 1)
        w2s_ref[...] = jnp.where(rows // HIDDEN_Q == cols, w2col,
                                 0.0).astype(jnp.bfloat16)
        b2s_ref[...] = jnp.transpose(pack_ref[2:3, :NUM_HASH])
        lb_ref[...] = jnp.zeros_like(lb_ref)

    block_t = x_ref.shape[0]
    lb_acc = jnp.zeros((NUM_EXPERTS,), jnp.float32)
    for m in range(0, block_t, _CHUNK_M):
        x = x_ref[m:m + _CHUNK_M, :].astype(jnp.bfloat16)
        h = jnp.dot(x, w1s_ref[...], preferred_element_type=jnp.float32)
        h = jnp.maximum(h + pack_ref[1:2, :], 0.0)
        hv = jnp.dot(h.astype(jnp.bfloat16), w2s_ref[...],
                     preferred_element_type=jnp.float32)
        hvT = jnp.transpose(hv) + b2s_ref[...]
        hash_ref[:, m:m + _CHUNK_M] = hvT
        summed = jnp.sum(hvT, axis=0, keepdims=True)
        e = summed.astype(jnp.int32)
        r = jnp.bitwise_and(e, NUM_EXPERTS - 1)
        ea_ref[m:m + _CHUNK_M] = r.reshape(_CHUNK_M)
        subl = jax.lax.broadcasted_iota(
            jnp.int32, (NUM_EXPERTS, _CHUNK_M), 0)
        rwT = (subl == r).astype(jnp.float32)
        rw_ref[:, m:m + _CHUNK_M] = rwT
        lb_acc = lb_acc + jnp.sum(rwT, axis=1)
    lb_ref[...] += lb_acc


@functools.partial(jax.jit, static_argnames=("block_t",))
def _run(hidden_flat, W1t, pack, block_t):
    T = hidden_flat.shape[0]
    grid = (T // block_t,)
    rwT, ea, hashesT, lb = pl.pallas_call(
        _routing_kernel,
        grid=grid,
        in_specs=[
            pl.BlockSpec((block_t, HIDDEN), lambda i: (i, 0)),
            pl.BlockSpec((NUM_HASH, HIDDEN_Q, HIDDEN), lambda i: (0, 0, 0)),
            pl.BlockSpec((3, HIDDEN), lambda i: (0, 0)),
        ],
        out_specs=[
            pl.BlockSpec((NUM_EXPERTS, block_t), lambda i: (0, i)),
            pl.BlockSpec((block_t,), lambda i: (i,)),
            pl.BlockSpec((NUM_HASH, block_t), lambda i: (0, i)),
            pl.BlockSpec((NUM_EXPERTS,), lambda i: (0,)),
        ],
        out_shape=[
            jax.ShapeDtypeStruct((NUM_EXPERTS, T), jnp.float32),
            jax.ShapeDtypeStruct((T,), jnp.int32),
            jax.ShapeDtypeStruct((NUM_HASH, T), jnp.float32),
            jax.ShapeDtypeStruct((NUM_EXPERTS,), jnp.float32),
        ],
        scratch_shapes=[
            pltpu.VMEM((HIDDEN, NUM_HASH * HIDDEN_Q), jnp.bfloat16),
            pltpu.VMEM((NUM_HASH * HIDDEN_Q, NUM_HASH), jnp.bfloat16),
            pltpu.VMEM((NUM_HASH, 1), jnp.float32),
        ],
    )(hidden_flat, W1t, pack)
    return rwT, ea, hashesT, lb


def kernel(hidden_states, W1, b1, W2, b2):
    B, S, H = hidden_states.shape
    hidden_flat = hidden_states.reshape(-1, H)
    W1t = jnp.transpose(W1, (0, 2, 1))
    pack = jnp.concatenate([
        W2.reshape(1, H),
        b1.reshape(1, H),
        jnp.pad(b2.reshape(1, NUM_HASH), ((0, 0), (0, H - NUM_HASH))),
    ], axis=0)
    rwT, ea, hashesT, lb = _run(hidden_flat, W1t, pack, block_t=4096)
    return rwT.T, ea[:, None], hashesT.T, lb
